# Initial kernel scaffold; baseline (speedup 1.0000x reference)
import jax
import jax.numpy as jnp
from jax.experimental import pallas as pl


def kernel(input_data, mask):
    raise NotImplementedError("write your pallas kernel here")


# speedup vs baseline: 4.7095x; 4.7095x over previous
import functools

import jax
import jax.numpy as jnp
from jax import lax
from jax.experimental import pallas as pl
from jax.experimental.pallas import tpu as pltpu
from jax.experimental.pallas import tpu_sc as plsc

B = 32
IMG = 256
NPIX = IMG * IMG
WIN = 251
NPATCH = 15876
K = 36
OUT_FLAT = NPATCH * K
CHUNK = 9072
NCHUNK = OUT_FLAT // CHUNK
L = 16
VECS = CHUNK // L


def _index_tables(mask):
    rows = (mask // WIN).astype(jnp.int32)
    cols = (mask % WIN).astype(jnp.int32)
    k = jnp.arange(K, dtype=jnp.int32)
    ki = k // 6
    kj = k % 6
    cbase = ((rows[:252, None] + ki[None, :]) * IMG
             + cols[:252, None] + kj[None, :]).reshape(-1)
    choff = ((rows[::252] - rows[0]) * IMG).astype(jnp.int32)
    choff = jnp.concatenate([choff, jnp.zeros((17,), jnp.int32)])
    return cbase, choff


_MESH = plsc.VectorSubcoreMesh(core_axis_name="c", subcore_axis_name="s")


@functools.partial(
    pl.kernel,
    mesh=_MESH,
    compiler_params=pltpu.CompilerParams(needs_layout_passes=False),
    out_type=jax.ShapeDtypeStruct((B * OUT_FLAT,), jnp.float32),
    scratch_types=[
        pltpu.VMEM((NPIX,), jnp.float32),
        pltpu.VMEM((CHUNK,), jnp.int32),
        pltpu.VMEM((80,), jnp.int32),
        pltpu.VMEM((CHUNK,), jnp.float32),
        pltpu.VMEM((CHUNK,), jnp.float32),
        pltpu.SemaphoreType.DMA,
        pltpu.SemaphoreType.DMA,
        pltpu.SemaphoreType.DMA,
    ],
)
def _unfold(x_hbm, cbase_hbm, choff_hbm, out_hbm,
            x_v, cbase_v, choff_v, buf0, buf1, xsem, sem0, sem1):
    b = lax.axis_index("s") * 2 + lax.axis_index("c")
    obase = b * OUT_FLAT
    pltpu.async_copy(x_hbm.at[pl.ds(b * NPIX, NPIX)], x_v, xsem)
    pltpu.sync_copy(cbase_hbm, cbase_v)
    pltpu.sync_copy(choff_hbm, choff_v)
    pltpu.make_async_copy(x_hbm.at[pl.ds(b * NPIX, NPIX)], x_v, xsem).wait()

    def do_chunk(c, buf, sem):
        @pl.when(c >= 2)
        def _wait_prev():
            pltpu.make_async_copy(
                buf, out_hbm.at[pl.ds(obase + (c - 2) * CHUNK, CHUNK)],
                sem).wait()

        choff_vec = choff_v[pl.ds(c, L)]
        choff = jnp.full((L,), choff_vec[0], jnp.int32)

        def inner(v, carry):
            idx = cbase_v[pl.ds(v * L, L)] + choff
            buf[pl.ds(v * L, L)] = plsc.load_gather(x_v, [idx])
            return carry

        lax.fori_loop(0, VECS, inner, 0)
        pltpu.async_copy(
            buf, out_hbm.at[pl.ds(obase + c * CHUNK, CHUNK)], sem)

    def body(c, carry):
        @pl.when(c % 2 == 0)
        def _even():
            do_chunk(c, buf0, sem0)

        @pl.when(c % 2 == 1)
        def _odd():
            do_chunk(c, buf1, sem1)

        return carry

    lax.fori_loop(0, NCHUNK, body, 0)
    pltpu.make_async_copy(
        buf0, out_hbm.at[pl.ds(obase + (NCHUNK - 1) * CHUNK, CHUNK)],
        sem0).wait()
    pltpu.make_async_copy(
        buf1, out_hbm.at[pl.ds(obase + (NCHUNK - 2) * CHUNK, CHUNK)],
        sem1).wait()


def kernel(input_data, mask):
    x_flat = input_data.reshape(B * NPIX)
    cbase, choff = _index_tables(mask)
    out = _unfold(x_flat, cbase, choff)
    return out.reshape(B, NPATCH, K)


# speedup vs baseline: 5.1695x; 1.0977x over previous
import functools

import jax
import jax.numpy as jnp
from jax import lax
from jax.experimental import pallas as pl
from jax.experimental.pallas import tpu as pltpu
from jax.experimental.pallas import tpu_sc as plsc

B = 32
IMG = 256
NPIX = IMG * IMG
WIN = 251
NPATCH = 15876
K = 36
OUT_FLAT = NPATCH * K
CHUNK = 9072
NCHUNK = OUT_FLAT // CHUNK
L = 16
VECS = CHUNK // L


def _index_tables(mask):
    rows = (mask // WIN).astype(jnp.int32)
    cols = (mask % WIN).astype(jnp.int32)
    k = jnp.arange(K, dtype=jnp.int32)
    ki = k // 6
    kj = k % 6
    cbase = ((rows[:252, None] + ki[None, :]) * IMG
             + cols[:252, None] + kj[None, :]).reshape(-1)
    choff = ((rows[::252] - rows[0]) * IMG).astype(jnp.int32)
    choff = jnp.concatenate([choff, jnp.zeros((17,), jnp.int32)])
    return cbase, choff


_MESH = plsc.VectorSubcoreMesh(core_axis_name="c", subcore_axis_name="s")


@functools.partial(
    pl.kernel,
    mesh=_MESH,
    compiler_params=pltpu.CompilerParams(needs_layout_passes=False),
    out_type=jax.ShapeDtypeStruct((B * OUT_FLAT,), jnp.float32),
    scratch_types=[
        pltpu.VMEM((NPIX,), jnp.float32),
        pltpu.VMEM((CHUNK,), jnp.int32),
        pltpu.VMEM((80,), jnp.int32),
        pltpu.VMEM((CHUNK,), jnp.float32),
        pltpu.VMEM((CHUNK,), jnp.float32),
        pltpu.SemaphoreType.DMA,
        pltpu.SemaphoreType.DMA,
        pltpu.SemaphoreType.DMA,
    ],
)
def _unfold(x_hbm, cbase_hbm, choff_hbm, out_hbm,
            x_v, cbase_v, choff_v, buf0, buf1, xsem, sem0, sem1):
    b = lax.axis_index("s") * 2 + lax.axis_index("c")
    obase = b * OUT_FLAT
    pltpu.async_copy(x_hbm.at[pl.ds(b * NPIX, NPIX)], x_v, xsem)
    pltpu.sync_copy(cbase_hbm, cbase_v)
    pltpu.sync_copy(choff_hbm, choff_v)
    pltpu.make_async_copy(x_hbm.at[pl.ds(b * NPIX, NPIX)], x_v, xsem).wait()

    def do_chunk(c, buf, sem):
        @pl.when(c >= 2)
        def _wait_prev():
            pltpu.make_async_copy(
                buf, out_hbm.at[pl.ds(obase + (c - 2) * CHUNK, CHUNK)],
                sem).wait()

        choff_vec = choff_v[pl.ds(c, L)]
        choff = jnp.full((L,), choff_vec[0], jnp.int32)

        @plsc.parallel_loop(0, VECS, unroll=8)
        def _gather(v):
            idx = cbase_v[pl.ds(v * L, L)] + choff
            buf[pl.ds(v * L, L)] = plsc.load_gather(x_v, [idx])
        pltpu.async_copy(
            buf, out_hbm.at[pl.ds(obase + c * CHUNK, CHUNK)], sem)

    def body(c, carry):
        @pl.when(c % 2 == 0)
        def _even():
            do_chunk(c, buf0, sem0)

        @pl.when(c % 2 == 1)
        def _odd():
            do_chunk(c, buf1, sem1)

        return carry

    lax.fori_loop(0, NCHUNK, body, 0)
    pltpu.make_async_copy(
        buf0, out_hbm.at[pl.ds(obase + (NCHUNK - 1) * CHUNK, CHUNK)],
        sem0).wait()
    pltpu.make_async_copy(
        buf1, out_hbm.at[pl.ds(obase + (NCHUNK - 2) * CHUNK, CHUNK)],
        sem1).wait()


def kernel(input_data, mask):
    x_flat = input_data.reshape(B * NPIX)
    cbase, choff = _index_tables(mask)
    out = _unfold(x_flat, cbase, choff)
    return out.reshape(B, NPATCH, K)


# speedup vs baseline: 17.3043x; 3.3474x over previous
import functools

import jax
import jax.numpy as jnp
from jax import lax
from jax.experimental import pallas as pl
from jax.experimental.pallas import tpu as pltpu
from jax.experimental.pallas import tpu_sc as plsc

B = 32
IMG = 256
NPIX = IMG * IMG
WIN = 251
NPATCH = 15876
K = 36
CP = 504
CHUNK = CP * K
SUB = 56
SUBK = SUB * K
SGROUPS = SUBK // 144
NSUB = NPATCH // SUB
SPP = CP // SUB
TP = NPATCH - NSUB * SUB
TGROUPS = TP * K // 144
L = 16


def _index_tables(mask):
    rows = (mask // WIN).astype(jnp.int32)
    cols = (mask % WIN).astype(jnp.int32)
    k = jnp.arange(K, dtype=jnp.int32)
    ki = k // 6
    kj = k % 6
    cbase = ((rows[:CP, None] + ki[None, :]) * IMG
             + cols[:CP, None] + kj[None, :]).reshape(-1)
    choff = ((rows[::CP] - rows[0]) * IMG).astype(jnp.int32)
    choff = jnp.concatenate([choff, jnp.zeros((16,), jnp.int32)])
    n = jnp.arange(144, dtype=jnp.int32)
    return cbase, choff, n // K, n % K


_MESH = plsc.VectorSubcoreMesh(core_axis_name="c", subcore_axis_name="s")


@functools.partial(
    pl.kernel,
    mesh=_MESH,
    compiler_params=pltpu.CompilerParams(needs_layout_passes=False),
    out_type=jax.ShapeDtypeStruct((B, NPATCH, K), jnp.float32),
    scratch_types=[
        pltpu.VMEM((NPIX,), jnp.float32),
        pltpu.VMEM((CHUNK,), jnp.int32),
        pltpu.VMEM((48,), jnp.int32),
        pltpu.VMEM((144,), jnp.int32),
        pltpu.VMEM((144,), jnp.int32),
        pltpu.VMEM((3, SUB, K), jnp.float32),
        pltpu.VMEM((TP, K), jnp.float32),
        pltpu.VMEM((SUBK,), jnp.float32),
        pltpu.SemaphoreType.DMA,
        pltpu.SemaphoreType.DMA((3,)),
    ],
)
def _unfold(x_hbm, cbase_hbm, choff_hbm, dr_hbm, dc_hbm, out_hbm,
            x_v, cbase_v, choff_v, dr_v, dc_v, ring, tailbuf, dummy_v,
            xsem, sems):
    b = lax.axis_index("s") * 2 + lax.axis_index("c")
    pltpu.async_copy(x_hbm.at[pl.ds(b * NPIX, NPIX)], x_v, xsem)
    pltpu.sync_copy(cbase_hbm, cbase_v)
    pltpu.sync_copy(choff_hbm, choff_v)
    pltpu.sync_copy(dr_hbm, dr_v)
    pltpu.sync_copy(dc_hbm, dc_v)
    drs = [dr_v[pl.ds(q * L, L)] for q in range(9)]
    dcs = [dc_v[pl.ds(q * L, L)] for q in range(9)]
    pltpu.make_async_copy(x_hbm.at[pl.ds(b * NPIX, NPIX)], x_v, xsem).wait()

    def drain_sub(s):
        i = lax.rem(s, 3)
        pltpu.make_async_copy(
            ring.at[i], out_hbm.at[b, pl.ds(s * SUB, SUB), :],
            sems.at[i]).wait()

    def gather_vecs(store, choff, base_off, ngroups):
        @plsc.parallel_loop(0, ngroups, unroll=2)
        def _gather(m):
            m4 = jnp.full((L,), 4 * m, jnp.int32)
            for q in range(9):
                idx = cbase_v[pl.ds(base_off + m * 144 + q * L, L)] + choff
                vals = plsc.load_gather(x_v, [idx])
                store(q, m4, vals)

    def sub_choff(s):
        vec = choff_v[pl.ds(s // SPP, L)]
        return jnp.full((L,), vec[0], jnp.int32)

    def body(s, carry):
        i = lax.rem(s, 3)

        @pl.when(s >= 3)
        def _wait_prev():
            drain_sub(s - 3)

        ivec = jnp.full((L,), i, jnp.int32)
        gather_vecs(
            lambda q, m4, vals: plsc.store_scatter(
                ring, [ivec, drs[q] + m4, dcs[q]], vals),
            sub_choff(s), lax.rem(s, SPP) * SUBK, SGROUPS)
        pltpu.async_copy(
            ring.at[i], out_hbm.at[b, pl.ds(s * SUB, SUB), :], sems.at[i])
        return carry

    lax.fori_loop(0, NSUB, body, 0)

    gather_vecs(
        lambda q, m4, vals: plsc.store_scatter(
            tailbuf, [drs[q] + m4, dcs[q]], vals),
        sub_choff(NSUB), (NSUB % SPP) * SUBK, TGROUPS)
    tail_dst = out_hbm.at[b, pl.ds(NSUB * SUB, TP), :]
    pltpu.async_copy(tailbuf, tail_dst, xsem)

    def drain(j, carry):
        drain_sub(NSUB - 3 + j)
        return carry

    lax.fori_loop(0, 3, drain, 0)
    pltpu.make_async_copy(tailbuf, tail_dst, xsem).wait()


def kernel(input_data, mask):
    x_flat = input_data.reshape(B * NPIX)
    cbase, choff, dr, dc = _index_tables(mask)
    return _unfold(x_flat, cbase, choff, dr, dc)


# speedup vs baseline: 17.3152x; 1.0006x over previous
import functools

import jax
import jax.numpy as jnp
from jax import lax
from jax.experimental import pallas as pl
from jax.experimental.pallas import tpu as pltpu
from jax.experimental.pallas import tpu_sc as plsc

B = 32
IMG = 256
NPIX = IMG * IMG
WIN = 251
NPATCH = 15876
K = 36
CP = 504
CHUNK = CP * K
SUB = 56
SUBK = SUB * K
SGROUPS = SUBK // 144
NSUB = NPATCH // SUB
SPP = CP // SUB
TP = NPATCH - NSUB * SUB
TGROUPS = TP * K // 144
L = 16


def _index_tables(mask):
    rows = (mask // WIN).astype(jnp.int32)
    cols = (mask % WIN).astype(jnp.int32)
    k = jnp.arange(K, dtype=jnp.int32)
    ki = k // 6
    kj = k % 6
    cbase = ((rows[:CP, None] + ki[None, :]) * IMG
             + cols[:CP, None] + kj[None, :]).reshape(-1)
    choff = ((rows[::CP] - rows[0]) * IMG).astype(jnp.int32)
    choff = jnp.concatenate([choff, jnp.zeros((16,), jnp.int32)])
    n = jnp.arange(144, dtype=jnp.int32)
    return cbase, choff, n // K, n % K


_MESH = plsc.VectorSubcoreMesh(core_axis_name="c", subcore_axis_name="s")


@functools.partial(
    pl.kernel,
    mesh=_MESH,
    compiler_params=pltpu.CompilerParams(
        needs_layout_passes=False, use_tc_tiling_on_sc=True),
    out_type=jax.ShapeDtypeStruct((B, NPATCH, K), jnp.float32),
    scratch_types=[
        pltpu.VMEM((NPIX,), jnp.float32),
        pltpu.VMEM((CHUNK,), jnp.int32),
        pltpu.VMEM((48,), jnp.int32),
        pltpu.VMEM((144,), jnp.int32),
        pltpu.VMEM((144,), jnp.int32),
        pltpu.VMEM((3, SUB, K), jnp.float32),
        pltpu.VMEM((TP, K), jnp.float32),
        pltpu.VMEM((SUBK,), jnp.float32),
        pltpu.SemaphoreType.DMA,
        pltpu.SemaphoreType.DMA((3,)),
    ],
)
def _unfold(x_hbm, cbase_hbm, choff_hbm, dr_hbm, dc_hbm, out_hbm,
            x_v, cbase_v, choff_v, dr_v, dc_v, ring, tailbuf, dummy_v,
            xsem, sems):
    b = lax.axis_index("s") * 2 + lax.axis_index("c")
    pltpu.async_copy(x_hbm.at[pl.ds(b * NPIX, NPIX)], x_v, xsem)
    pltpu.sync_copy(cbase_hbm, cbase_v)
    pltpu.sync_copy(choff_hbm, choff_v)
    pltpu.sync_copy(dr_hbm, dr_v)
    pltpu.sync_copy(dc_hbm, dc_v)
    drs = [dr_v[pl.ds(q * L, L)] for q in range(9)]
    dcs = [dc_v[pl.ds(q * L, L)] for q in range(9)]
    pltpu.make_async_copy(x_hbm.at[pl.ds(b * NPIX, NPIX)], x_v, xsem).wait()

    def drain_sub(s):
        i = lax.rem(s, 3)
        pltpu.make_async_copy(
            ring.at[i], out_hbm.at[b, pl.ds(s * SUB, SUB), :],
            sems.at[i]).wait()

    def gather_vecs(store, choff, base_off, ngroups):
        @plsc.parallel_loop(0, ngroups, unroll=2)
        def _gather(m):
            m4 = jnp.full((L,), 4 * m, jnp.int32)
            for q in range(9):
                idx = cbase_v[pl.ds(base_off + m * 144 + q * L, L)] + choff
                vals = plsc.load_gather(x_v, [idx])
                store(q, m4, vals)

    def sub_choff(s):
        vec = choff_v[pl.ds(s // SPP, L)]
        return jnp.full((L,), vec[0], jnp.int32)

    def body(s, carry):
        i = lax.rem(s, 3)

        @pl.when(s >= 3)
        def _wait_prev():
            drain_sub(s - 3)

        ivec = jnp.full((L,), i, jnp.int32)
        gather_vecs(
            lambda q, m4, vals: plsc.store_scatter(
                ring, [ivec, drs[q] + m4, dcs[q]], vals),
            sub_choff(s), lax.rem(s, SPP) * SUBK, SGROUPS)
        pltpu.async_copy(
            ring.at[i], out_hbm.at[b, pl.ds(s * SUB, SUB), :], sems.at[i])
        return carry

    lax.fori_loop(0, NSUB, body, 0)

    gather_vecs(
        lambda q, m4, vals: plsc.store_scatter(
            tailbuf, [drs[q] + m4, dcs[q]], vals),
        sub_choff(NSUB), (NSUB % SPP) * SUBK, TGROUPS)
    tail_dst = out_hbm.at[b, pl.ds(NSUB * SUB, TP), :]
    pltpu.async_copy(tailbuf, tail_dst, xsem)

    def drain(j, carry):
        drain_sub(NSUB - 3 + j)
        return carry

    lax.fori_loop(0, 3, drain, 0)
    pltpu.make_async_copy(tailbuf, tail_dst, xsem).wait()


def kernel(input_data, mask):
    x_flat = input_data.reshape(B * NPIX)
    cbase, choff, dr, dc = _index_tables(mask)
    return _unfold(x_flat, cbase, choff, dr, dc)
